# Initial kernel scaffold; baseline (speedup 1.0000x reference)
import jax
import jax.numpy as jnp
from jax import lax
from jax.experimental import pallas as pl

_E = 8
_K = 2
_Z_LOSS_COEF = 0.01
_AUX_LOSS_COEF = 0.01


def _router_kernel(x_ref, wr_ref, comb_ref, aux_ref, xbf_ref):
    x = x_ref[...]
    logits = jnp.dot(x, wr_ref[...], preferred_element_type=jnp.float32)
    mx = jnp.max(logits, axis=-1, keepdims=True)
    ex = jnp.exp(logits - mx)
    den = jnp.sum(ex, axis=-1, keepdims=True)
    probs = ex / den

    T, E = logits.shape
    iota = lax.broadcasted_iota(jnp.int32, (T, E), 1)
    m1 = jnp.max(probs, axis=-1, keepdims=True)
    idx1 = jnp.min(jnp.where(probs == m1, iota, E), axis=-1, keepdims=True)
    sel1 = iota == idx1
    pm = jnp.where(sel1, -1.0, probs)
    m2 = jnp.max(pm, axis=-1, keepdims=True)
    idx2 = jnp.min(jnp.where(pm == m2, iota, E), axis=-1, keepdims=True)
    sel2 = iota == idx2
    s = m1 + m2
    comb = jnp.where(sel1, m1 / s, 0.0) + jnp.where(sel2, m2 / s, 0.0)
    comb_ref[...] = comb

    usage = jnp.mean(probs, axis=0, keepdims=True)
    selection = (
        jnp.mean(sel1.astype(jnp.float32) + sel2.astype(jnp.float32), axis=0,
                 keepdims=True) / _K)
    lb = E * jnp.sum(usage * selection)
    lse = jnp.log(den) + mx
    z = jnp.mean(lse * lse)
    aux_ref[0, 0] = _AUX_LOSS_COEF * lb + _Z_LOSS_COEF * z

    xbf_ref[...] = x.astype(jnp.bfloat16)


def _mlp_kernel(xbf_ref, w1_ref, b1_ref, w2_ref, b2_ref, comb_ref, out_ref):
    e = pl.program_id(1)
    h = pl.program_id(2)

    @pl.when((e == 0) & (h == 0))
    def _():
        out_ref[...] = jnp.zeros_like(out_ref)

    xb = xbf_ref[...]
    w1 = w1_ref[0].astype(jnp.bfloat16)
    hpre = jnp.dot(xb, w1, preferred_element_type=jnp.float32) + b1_ref[0]
    hact = 0.5 * hpre * (1.0 + lax.erf(hpre * 0.7071067811865476))

    B = xb.shape[0]
    iota = lax.broadcasted_iota(jnp.int32, (B, _E), 1)
    c = jnp.sum(jnp.where(iota == e, comb_ref[...], 0.0), axis=1, keepdims=True)

    hs = (hact * c).astype(jnp.bfloat16)
    w2 = w2_ref[0].astype(jnp.bfloat16)
    out_ref[...] += jnp.dot(hs, w2, preferred_element_type=jnp.float32)

    @pl.when(h == 0)
    def _():
        out_ref[...] += c * b2_ref[0]


def kernel(x, Wr, W1, b1, W2, b2):
    T, D = x.shape
    E = Wr.shape[1]
    H = W1.shape[2]

    comb, aux, xbf = pl.pallas_call(
        _router_kernel,
        out_shape=[
            jax.ShapeDtypeStruct((T, E), jnp.float32),
            jax.ShapeDtypeStruct((1, 1), jnp.float32),
            jax.ShapeDtypeStruct((T, D), jnp.bfloat16),
        ],
    )(x, Wr)

    Hb = 512
    nh = H // Hb
    b1r = b1.reshape(E, 1, H)
    b2r = b2.reshape(E, 1, D)

    out = pl.pallas_call(
        _mlp_kernel,
        grid=(1, E, nh),
        in_specs=[
            pl.BlockSpec((T, D), lambda t, e, h: (0, 0)),
            pl.BlockSpec((1, D, Hb), lambda t, e, h: (e, 0, h)),
            pl.BlockSpec((1, 1, Hb), lambda t, e, h: (e, 0, h)),
            pl.BlockSpec((1, Hb, D), lambda t, e, h: (e, h, 0)),
            pl.BlockSpec((1, 1, D), lambda t, e, h: (e, 0, 0)),
            pl.BlockSpec((T, E), lambda t, e, h: (0, 0)),
        ],
        out_specs=pl.BlockSpec((T, D), lambda t, e, h: (0, 0)),
        out_shape=jax.ShapeDtypeStruct((T, D), jnp.float32),
    )(xbf, W1, b1r, W2, b2r, comb)

    return out, aux[0, 0]


# speedup vs baseline: 3.2077x; 3.2077x over previous
import jax
import jax.numpy as jnp
from jax import lax
from jax.experimental import pallas as pl

_E = 8
_K = 2
_Z_LOSS_COEF = 0.01
_AUX_LOSS_COEF = 0.01


def _router_kernel(x_ref, wr_ref, comb_ref, aux_ref, xbf_ref):
    x = x_ref[...]
    logits = jnp.dot(x, wr_ref[...], preferred_element_type=jnp.float32)
    mx = jnp.max(logits, axis=-1, keepdims=True)
    ex = jnp.exp(logits - mx)
    den = jnp.sum(ex, axis=-1, keepdims=True)
    probs = ex / den

    T, E = logits.shape
    iota = lax.broadcasted_iota(jnp.int32, (T, E), 1)
    m1 = jnp.max(probs, axis=-1, keepdims=True)
    idx1 = jnp.min(jnp.where(probs == m1, iota, E), axis=-1, keepdims=True)
    sel1 = iota == idx1
    pm = jnp.where(sel1, -1.0, probs)
    m2 = jnp.max(pm, axis=-1, keepdims=True)
    idx2 = jnp.min(jnp.where(pm == m2, iota, E), axis=-1, keepdims=True)
    sel2 = iota == idx2
    s = m1 + m2
    comb = jnp.where(sel1, m1 / s, 0.0) + jnp.where(sel2, m2 / s, 0.0)
    comb_ref[...] = comb

    usage = jnp.mean(probs, axis=0, keepdims=True)
    selection = (
        jnp.mean(sel1.astype(jnp.float32) + sel2.astype(jnp.float32), axis=0,
                 keepdims=True) / _K)
    lb = E * jnp.sum(usage * selection)
    lse = jnp.log(den) + mx
    z = jnp.mean(lse * lse)
    aux_ref[...] = jnp.reshape(_AUX_LOSS_COEF * lb + _Z_LOSS_COEF * z, (1, 1))

    xbf_ref[...] = x.astype(jnp.bfloat16)


def _mlp_kernel(xbf_ref, w1_ref, b1_ref, w2_ref, b2_ref, comb_ref, out_ref):
    e = pl.program_id(1)
    h = pl.program_id(2)

    @pl.when((e == 0) & (h == 0))
    def _():
        out_ref[...] = jnp.zeros_like(out_ref)

    xb = xbf_ref[...]
    w1 = w1_ref[0].astype(jnp.bfloat16)
    hpre = jnp.dot(xb, w1, preferred_element_type=jnp.float32) + b1_ref[0]
    hact = 0.5 * hpre * (1.0 + lax.erf(hpre * 0.7071067811865476))

    B = xb.shape[0]
    iota = lax.broadcasted_iota(jnp.int32, (B, _E), 1)
    c = jnp.sum(jnp.where(iota == e, comb_ref[...], 0.0), axis=1, keepdims=True)

    hs = (hact * c).astype(jnp.bfloat16)
    w2 = w2_ref[0].astype(jnp.bfloat16)
    out_ref[...] += jnp.dot(hs, w2, preferred_element_type=jnp.float32)

    @pl.when(h == 0)
    def _():
        out_ref[...] += c * b2_ref[0]


def kernel(x, Wr, W1, b1, W2, b2):
    T, D = x.shape
    E = Wr.shape[1]
    H = W1.shape[2]

    comb, aux, xbf = pl.pallas_call(
        _router_kernel,
        out_shape=[
            jax.ShapeDtypeStruct((T, E), jnp.float32),
            jax.ShapeDtypeStruct((1, 1), jnp.float32),
            jax.ShapeDtypeStruct((T, D), jnp.bfloat16),
        ],
    )(x, Wr)

    Hb = 512
    nh = H // Hb
    b1r = b1.reshape(E, 1, H)
    b2r = b2.reshape(E, 1, D)

    out = pl.pallas_call(
        _mlp_kernel,
        grid=(1, E, nh),
        in_specs=[
            pl.BlockSpec((T, D), lambda t, e, h: (0, 0)),
            pl.BlockSpec((1, D, Hb), lambda t, e, h: (e, 0, h)),
            pl.BlockSpec((1, 1, Hb), lambda t, e, h: (e, 0, h)),
            pl.BlockSpec((1, Hb, D), lambda t, e, h: (e, h, 0)),
            pl.BlockSpec((1, 1, D), lambda t, e, h: (e, 0, 0)),
            pl.BlockSpec((T, E), lambda t, e, h: (0, 0)),
        ],
        out_specs=pl.BlockSpec((T, D), lambda t, e, h: (0, 0)),
        out_shape=jax.ShapeDtypeStruct((T, D), jnp.float32),
    )(xbf, W1, b1r, W2, b2r, comb)

    return out, aux[0, 0]
